# baseline (speedup 1.0000x reference)
import jax
import jax.numpy as jnp
from jax.experimental import pallas as pl
from jax.experimental.pallas import tpu as pltpu

_A = 64
_F = 32
_H = 64
_NRBF = 64
_GAMMA = 10.0
_CUTOFF = 10.0
_NI = 3
_MPP = 2

_bf16 = jnp.bfloat16
_f32 = jnp.float32


def _bdot(a16, b16):
    return jax.lax.dot_general(a16, b16, (((1,), (0,)), ((), ())),
                               preferred_element_type=_f32)


def _schnet_kernel(X_ref, D_ref, C2_ref, We_ref, be_ref, Wr_ref, br_ref,
                   Wp_ref, bp_ref, Wa1_ref, ba1_ref, Wa2_ref, ba2_ref,
                   Wo1_ref, bo1_ref, Wo2_ref, bo2_ref, y_ref):
    lane = jax.lax.broadcasted_iota(jnp.int32, (1, 1, _MPP * _NRBF), 2)
    in_first = lane < _NRBF

    d_a = D_ref[0]
    d_b = D_ref[1]
    dsel = jnp.where(in_first, d_a[:, :, None], d_b[:, :, None])
    rbf = jnp.exp(-_GAMMA * jnp.square(dsel - C2_ref[...]))
    flat = rbf.reshape(_A * _A, _MPP * _NRBF)
    rbf16_a = flat[:, :_NRBF].astype(_bf16)
    rbf16_b = flat[:, _NRBF:].astype(_bf16)

    X2 = X_ref[...].reshape(_MPP * _A, _F)
    h = _bdot(X2.astype(_bf16), We_ref[...].astype(_bf16)) + be_ref[...]

    for i in range(_NI):
        Wr16 = Wr_ref[i].astype(_bf16)
        Wp16 = Wp_ref[i].astype(_bf16)
        aggs = []
        for rbf16 in (rbf16_a, rbf16_b):
            em = _bdot(rbf16, Wr16) + br_ref[i:i + 1, :]
            em = _bdot(em.astype(_bf16), Wp16) + bp_ref[i:i + 1, :]
            aggs.append(jnp.sum(em.reshape(_A, _A, _H), axis=0))
        agg = jnp.concatenate(aggs, axis=0)
        pre = _bdot(agg.astype(_bf16), Wa1_ref[i].astype(_bf16)) \
            + ba1_ref[i:i + 1, :]
        t = pre * jax.nn.sigmoid(pre)
        t = _bdot(t.astype(_bf16), Wa2_ref[i].astype(_bf16)) \
            + ba2_ref[i:i + 1, :]
        h = h + t

    u = _bdot(h.astype(_bf16), Wo1_ref[...].astype(_bf16)) + bo1_ref[...]
    u = u * jax.nn.sigmoid(u)
    o = _bdot(u.astype(_bf16), Wo2_ref[...].astype(_bf16)) + bo2_ref[...]
    ya = jnp.sum(o[0:_A, :], axis=0, keepdims=True) / float(_A)
    yb = jnp.sum(o[_A:, :], axis=0, keepdims=True) / float(_A)
    y_ref[0] = jnp.concatenate([ya, yb], axis=0)


def kernel(X, R, batch, W_emb, b_emb, Wr, br, Wp, bp, Wa1, ba1, Wa2, ba2,
           Wo1, bo1, Wo2, bo2):
    Bn = X.shape[0]
    grid = (Bn // _MPP,)

    send = jnp.repeat(jnp.arange(_A), _A)
    recv = jnp.tile(jnp.arange(_A), _A)

    def _dmat(Ri):
        diff = Ri[send] - Ri[recv]
        d2 = jnp.sum(diff * diff, axis=1)
        return jnp.sqrt(jnp.maximum(d2, 1e-12)).reshape(_A, _A)

    D = jax.vmap(_dmat)(R)

    be = b_emb.reshape(1, _H)
    bo1r = bo1.reshape(1, _H)
    bo2r = bo2.reshape(1, 1)
    cvec = jnp.linspace(0.0, _CUTOFF, _NRBF)
    c2 = jnp.concatenate([cvec, cvec]).reshape(1, 1, _MPP * _NRBF)

    full2 = lambda b: (0, 0)
    full3 = lambda b: (0, 0, 0)
    out = pl.pallas_call(
        _schnet_kernel,
        grid=grid,
        in_specs=[
            pl.BlockSpec((_MPP, _A, _F), lambda b: (b, 0, 0)),
            pl.BlockSpec((_MPP, _A, _A), lambda b: (b, 0, 0)),
            pl.BlockSpec((1, 1, _MPP * _NRBF), full3),
            pl.BlockSpec((_F, _H), full2),
            pl.BlockSpec((1, _H), full2),
            pl.BlockSpec((_NI, _NRBF, _H), full3),
            pl.BlockSpec((_NI, _H), full2),
            pl.BlockSpec((_NI, _H, _H), full3),
            pl.BlockSpec((_NI, _H), full2),
            pl.BlockSpec((_NI, _H, _H), full3),
            pl.BlockSpec((_NI, _H), full2),
            pl.BlockSpec((_NI, _H, _H), full3),
            pl.BlockSpec((_NI, _H), full2),
            pl.BlockSpec((_H, _H), full2),
            pl.BlockSpec((1, _H), full2),
            pl.BlockSpec((_H, 1), full2),
            pl.BlockSpec((1, 1), full2),
        ],
        out_specs=pl.BlockSpec((1, _MPP, 1), lambda b: (b, 0, 0)),
        out_shape=jax.ShapeDtypeStruct((Bn // _MPP, _MPP, 1), jnp.float32),
        compiler_params=pltpu.CompilerParams(
            dimension_semantics=("arbitrary",)),
    )(X, D, c2, W_emb, be, Wr, br, Wp, bp, Wa1, ba1, Wa2, ba2, Wo1, bo1r,
      Wo2, bo2r)
    return out.reshape(Bn, 1)


# speedup vs baseline: 1.0011x; 1.0011x over previous
import jax
import jax.numpy as jnp
from jax.experimental import pallas as pl
from jax.experimental.pallas import tpu as pltpu

_A = 64
_F = 32
_H = 64
_NRBF = 64
_GAMMA = 10.0
_CUTOFF = 10.0
_NI = 3
_MPP = 2

_bf16 = jnp.bfloat16
_f32 = jnp.float32


def _bdot(a16, b16):
    return jax.lax.dot_general(a16, b16, (((1,), (0,)), ((), ())),
                               preferred_element_type=_f32)


def _schnet_kernel(X_ref, D_ref, C2_ref, We_ref, be_ref, Wr_ref, br_ref,
                   Wp_ref, bp_ref, Wa1_ref, ba1_ref, Wa2_ref, ba2_ref,
                   Wo1_ref, bo1_ref, Wo2_ref, bo2_ref, y_ref):
    lane = jax.lax.broadcasted_iota(jnp.int32, (1, 1, _MPP * _NRBF), 2)
    in_first = lane < _NRBF

    d_a = D_ref[0]
    d_b = D_ref[1]
    dsel = jnp.where(in_first, d_a[:, :, None], d_b[:, :, None])
    rbf = jnp.exp(-_GAMMA * jnp.square(dsel - C2_ref[...]))
    flat = rbf.reshape(_A * _A, _MPP * _NRBF)
    rbf16 = jnp.concatenate(
        [flat[:, :_NRBF], flat[:, _NRBF:]], axis=0).astype(_bf16)

    X2 = X_ref[...].reshape(_MPP * _A, _F)
    h = _bdot(X2.astype(_bf16), We_ref[...].astype(_bf16)) + be_ref[...]

    for i in range(_NI):
        em = _bdot(rbf16, Wr_ref[i].astype(_bf16)) + br_ref[i:i + 1, :]
        em = _bdot(em.astype(_bf16), Wp_ref[i].astype(_bf16)) \
            + bp_ref[i:i + 1, :]
        agg = jnp.concatenate(
            [jnp.sum(em[:_A * _A].reshape(_A, _A, _H), axis=0),
             jnp.sum(em[_A * _A:].reshape(_A, _A, _H), axis=0)],
            axis=0)
        pre = _bdot(agg.astype(_bf16), Wa1_ref[i].astype(_bf16)) \
            + ba1_ref[i:i + 1, :]
        t = pre * jax.nn.sigmoid(pre)
        t = _bdot(t.astype(_bf16), Wa2_ref[i].astype(_bf16)) \
            + ba2_ref[i:i + 1, :]
        h = h + t

    u = _bdot(h.astype(_bf16), Wo1_ref[...].astype(_bf16)) + bo1_ref[...]
    u = u * jax.nn.sigmoid(u)
    o = _bdot(u.astype(_bf16), Wo2_ref[...].astype(_bf16)) + bo2_ref[...]
    ya = jnp.sum(o[0:_A, :], axis=0, keepdims=True) / float(_A)
    yb = jnp.sum(o[_A:, :], axis=0, keepdims=True) / float(_A)
    y_ref[0] = jnp.concatenate([ya, yb], axis=0)


def kernel(X, R, batch, W_emb, b_emb, Wr, br, Wp, bp, Wa1, ba1, Wa2, ba2,
           Wo1, bo1, Wo2, bo2):
    Bn = X.shape[0]
    grid = (Bn // _MPP,)

    send = jnp.repeat(jnp.arange(_A), _A)
    recv = jnp.tile(jnp.arange(_A), _A)

    def _dmat(Ri):
        diff = Ri[send] - Ri[recv]
        d2 = jnp.sum(diff * diff, axis=1)
        return jnp.sqrt(jnp.maximum(d2, 1e-12)).reshape(_A, _A)

    D = jax.vmap(_dmat)(R)

    be = b_emb.reshape(1, _H)
    bo1r = bo1.reshape(1, _H)
    bo2r = bo2.reshape(1, 1)
    cvec = jnp.linspace(0.0, _CUTOFF, _NRBF)
    c2 = jnp.concatenate([cvec, cvec]).reshape(1, 1, _MPP * _NRBF)

    full2 = lambda b: (0, 0)
    full3 = lambda b: (0, 0, 0)
    out = pl.pallas_call(
        _schnet_kernel,
        grid=grid,
        in_specs=[
            pl.BlockSpec((_MPP, _A, _F), lambda b: (b, 0, 0)),
            pl.BlockSpec((_MPP, _A, _A), lambda b: (b, 0, 0)),
            pl.BlockSpec((1, 1, _MPP * _NRBF), full3),
            pl.BlockSpec((_F, _H), full2),
            pl.BlockSpec((1, _H), full2),
            pl.BlockSpec((_NI, _NRBF, _H), full3),
            pl.BlockSpec((_NI, _H), full2),
            pl.BlockSpec((_NI, _H, _H), full3),
            pl.BlockSpec((_NI, _H), full2),
            pl.BlockSpec((_NI, _H, _H), full3),
            pl.BlockSpec((_NI, _H), full2),
            pl.BlockSpec((_NI, _H, _H), full3),
            pl.BlockSpec((_NI, _H), full2),
            pl.BlockSpec((_H, _H), full2),
            pl.BlockSpec((1, _H), full2),
            pl.BlockSpec((_H, 1), full2),
            pl.BlockSpec((1, 1), full2),
        ],
        out_specs=pl.BlockSpec((1, _MPP, 1), lambda b: (b, 0, 0)),
        out_shape=jax.ShapeDtypeStruct((Bn // _MPP, _MPP, 1), jnp.float32),
        compiler_params=pltpu.CompilerParams(
            dimension_semantics=("parallel",)),
    )(X, D, c2, W_emb, be, Wr, br, Wp, bp, Wa1, ba1, Wa2, ba2, Wo1, bo1r,
      Wo2, bo2r)
    return out.reshape(Bn, 1)
